# baseline (device time: 59345 ns/iter reference)
import jax
import jax.numpy as jnp
from jax import lax
from jax.experimental import pallas as pl
from jax.experimental.pallas import tpu as pltpu

N_DEV = 4
B = 2
SQ_LOC = 128
D_MODEL = 512
HQ = 16
DH = 64
SKV = 128
WQ_COLS = 256
WO_ROWS = 256


def kernel(x, Wq, K_ext, V_ext, Wo):
    def body(x_ref, wq_ref, k_ref, v_ref, wo_ref, out_ref,
             wq_buf, wo_buf, q_ref, ctx_ref,
             wq_send_sems, wq_recv_sems, wo_send_sems, wo_recv_sems):
        my_pos = lax.axis_index("i")
        left = (my_pos - 1) % N_DEV
        right = (my_pos + 1) % N_DEV

        barrier_sem = pltpu.get_barrier_semaphore()
        for nbr in (left, right):
            pl.semaphore_signal(
                barrier_sem, inc=1,
                device_id=(nbr,), device_id_type=pltpu.DeviceIdType.MESH,
            )
        pl.semaphore_wait(barrier_sem, 2)

        wq_buf[0] = wq_ref[...]
        wo_buf[0] = wo_ref[...]

        for h in range(N_DEV - 1):
            wq_rdma = pltpu.make_async_remote_copy(
                src_ref=wq_buf.at[h],
                dst_ref=wq_buf.at[h + 1],
                send_sem=wq_send_sems.at[h],
                recv_sem=wq_recv_sems.at[h + 1],
                device_id=(right,),
                device_id_type=pltpu.DeviceIdType.MESH,
            )
            wo_rdma = pltpu.make_async_remote_copy(
                src_ref=wo_buf.at[h],
                dst_ref=wo_buf.at[h + 1],
                send_sem=wo_send_sems.at[h],
                recv_sem=wo_recv_sems.at[h + 1],
                device_id=(right,),
                device_id_type=pltpu.DeviceIdType.MESH,
            )
            wq_rdma.start()
            wo_rdma.start()
            wq_rdma.wait()
            wo_rdma.wait()

        for b in range(B):
            xb = x_ref[b]
            for s in range(N_DEV):
                origin = (my_pos - s) % N_DEV
                q_ref[b, :, origin, :] = jnp.dot(
                    xb, wq_buf[s], preferred_element_type=jnp.float32
                )

        rows = lax.broadcasted_iota(jnp.int32, (SQ_LOC, SKV), 0)
        cols = lax.broadcasted_iota(jnp.int32, (SQ_LOC, SKV), 1)
        qb = rows // 64 + 2 * my_pos
        kb = cols // 64
        mask = (qb == kb) | (kb == 0) | ((qb + kb) % 3 == 0)

        for b in range(B):
            for h in range(HQ):
                g, off = h // 4, (h % 4) * DH
                q = q_ref[b, :, g, off:off + DH]
                k = k_ref[b, :, h, :]
                s = lax.dot_general(
                    q, k, (((1,), (1,)), ((), ())),
                    preferred_element_type=jnp.float32,
                ) * 0.125
                s = jnp.where(mask, s, -1e9)
                m = jnp.max(s, axis=1, keepdims=True)
                w = jnp.exp(s - m)
                w = w / jnp.sum(w, axis=1, keepdims=True)
                ctx_ref[b, :, g, off:off + DH] = jnp.dot(
                    w, v_ref[b, :, h, :], preferred_element_type=jnp.float32
                )

        for b in range(B):
            acc = jnp.zeros((SQ_LOC, D_MODEL), jnp.float32)
            for s in range(N_DEV):
                origin = (my_pos - s) % N_DEV
                acc = acc + jnp.dot(
                    ctx_ref[b, :, origin, :], wo_buf[s],
                    preferred_element_type=jnp.float32,
                )
            out_ref[b] = acc

    return pl.pallas_call(
        body,
        out_shape=jax.ShapeDtypeStruct((B, SQ_LOC, D_MODEL), jnp.float32),
        in_specs=[pl.BlockSpec(memory_space=pltpu.VMEM)] * 5,
        out_specs=pl.BlockSpec(memory_space=pltpu.VMEM),
        scratch_shapes=[
            pltpu.VMEM((N_DEV, D_MODEL, WQ_COLS), jnp.float32),
            pltpu.VMEM((N_DEV, WO_ROWS, D_MODEL), jnp.float32),
            pltpu.VMEM((B, SQ_LOC, N_DEV, WQ_COLS), jnp.float32),
            pltpu.VMEM((B, SQ_LOC, N_DEV, WO_ROWS), jnp.float32),
            pltpu.SemaphoreType.DMA((N_DEV,)),
            pltpu.SemaphoreType.DMA((N_DEV,)),
            pltpu.SemaphoreType.DMA((N_DEV,)),
            pltpu.SemaphoreType.DMA((N_DEV,)),
        ],
        compiler_params=pltpu.CompilerParams(collective_id=0),
    )(x, Wq, K_ext, V_ext, Wo)


# device time: 26662 ns/iter; 2.2258x vs baseline; 2.2258x over previous
import jax
import jax.numpy as jnp
from jax import lax
from jax.experimental import pallas as pl
from jax.experimental.pallas import tpu as pltpu

N_DEV = 4
B = 2
SQ_LOC = 128
D_MODEL = 512
HQ = 16
H_BLK = 4
DH = 64
SKV = 128
WQ_COLS = 256
WO_ROWS = 256


def kernel(x, Wq, K_ext, V_ext, Wo):
    def body(x_ref, wq_ref, k_ref, v_ref, wo_ref, out_ref,
             wq_bf, wo_bf, wq_gath, wo_gath, kt_ref, vt_ref,
             send_sems, recv_sems):
        my_pos = lax.axis_index("i")
        left = (my_pos - 1) % N_DEV
        right = (my_pos + 1) % N_DEV
        opp = (my_pos + 2) % N_DEV

        barrier_sem = pltpu.get_barrier_semaphore()
        for nbr in (left, right, opp):
            pl.semaphore_signal(
                barrier_sem, inc=1,
                device_id=(nbr,), device_id_type=pltpu.DeviceIdType.MESH,
            )
        pl.semaphore_wait(barrier_sem, 3)

        wq_bf[...] = wq_ref[...].astype(jnp.bfloat16)
        wo_bf[...] = wo_ref[...].astype(jnp.bfloat16)

        def push(src, gath, target, slot, sem):
            return pltpu.make_async_remote_copy(
                src_ref=src,
                dst_ref=gath.at[slot],
                send_sem=send_sems.at[sem],
                recv_sem=recv_sems.at[sem],
                device_id=(target,),
                device_id_type=pltpu.DeviceIdType.MESH,
            )

        wq_to_r = push(wq_bf, wq_gath, right, 0, 0)
        wq_to_l = push(wq_bf, wq_gath, left, 1, 1)
        wq_to_o = push(wq_bf, wq_gath, opp, 2, 2)
        wo_to_r = push(wo_bf, wo_gath, right, 0, 3)
        wo_to_l = push(wo_bf, wo_gath, left, 1, 4)
        wo_to_o = push(wo_bf, wo_gath, opp, 2, 5)
        for r in (wq_to_r, wq_to_l, wq_to_o, wo_to_r, wo_to_l, wo_to_o):
            r.start()

        for b in range(B):
            for h in range(HQ):
                kt_ref[b, h] = k_ref[b, :, h, :]
                vt_ref[b, h] = v_ref[b, :, h, :]

        rows = lax.broadcasted_iota(jnp.int32, (SQ_LOC, SKV), 0)
        cols = lax.broadcasted_iota(jnp.int32, (SQ_LOC, SKV), 1)
        qb = rows // 64 + 2 * my_pos
        kb = cols // 64
        mask = (qb == kb) | (kb == 0) | ((qb + kb) % 3 == 0)

        def block(origin, wq_c, wo_c):
            partials = []
            for b in range(B):
                qblk = jnp.dot(x_ref[b], wq_c,
                               preferred_element_type=jnp.float32)
                ctxs = []
                for j in range(H_BLK):
                    h = origin * H_BLK + j
                    q = qblk[:, j * DH:(j + 1) * DH]
                    k = kt_ref[b, h]
                    s = lax.dot_general(
                        q, k, (((1,), (1,)), ((), ())),
                        preferred_element_type=jnp.float32,
                    ) * 0.125
                    s = jnp.where(mask, s, -1e9)
                    m = jnp.max(s, axis=1, keepdims=True)
                    w = jnp.exp(s - m)
                    w = w / jnp.sum(w, axis=1, keepdims=True)
                    ctxs.append(jnp.dot(w, vt_ref[b, h],
                                        preferred_element_type=jnp.float32))
                ctx = jnp.concatenate(ctxs, axis=1)
                partials.append(jnp.dot(ctx, wo_c,
                                        preferred_element_type=jnp.float32))
            return partials

        acc = block(my_pos, wq_ref[...], wo_ref[...])

        wq_to_r.wait_recv()
        wo_to_r.wait_recv()
        p = block(left, wq_gath[0].astype(jnp.float32),
                  wo_gath[0].astype(jnp.float32))
        acc = [a + q for a, q in zip(acc, p)]

        wq_to_l.wait_recv()
        wo_to_l.wait_recv()
        p = block(right, wq_gath[1].astype(jnp.float32),
                  wo_gath[1].astype(jnp.float32))
        acc = [a + q for a, q in zip(acc, p)]

        wq_to_o.wait_recv()
        wo_to_o.wait_recv()
        p = block(opp, wq_gath[2].astype(jnp.float32),
                  wo_gath[2].astype(jnp.float32))
        acc = [a + q for a, q in zip(acc, p)]

        for b in range(B):
            out_ref[b] = acc[b]

        for r in (wq_to_r, wq_to_l, wq_to_o, wo_to_r, wo_to_l, wo_to_o):
            r.wait_send()

    return pl.pallas_call(
        body,
        out_shape=jax.ShapeDtypeStruct((B, SQ_LOC, D_MODEL), jnp.float32),
        in_specs=[pl.BlockSpec(memory_space=pltpu.VMEM)] * 5,
        out_specs=pl.BlockSpec(memory_space=pltpu.VMEM),
        scratch_shapes=[
            pltpu.VMEM((D_MODEL, WQ_COLS), jnp.bfloat16),
            pltpu.VMEM((WO_ROWS, D_MODEL), jnp.bfloat16),
            pltpu.VMEM((3, D_MODEL, WQ_COLS), jnp.bfloat16),
            pltpu.VMEM((3, WO_ROWS, D_MODEL), jnp.bfloat16),
            pltpu.VMEM((B, HQ, SKV, DH), jnp.float32),
            pltpu.VMEM((B, HQ, SKV, DH), jnp.float32),
            pltpu.SemaphoreType.DMA((6,)),
            pltpu.SemaphoreType.DMA((6,)),
        ],
        compiler_params=pltpu.CompilerParams(collective_id=0),
    )(x, Wq, K_ext, V_ext, Wo)


# device time: 24028 ns/iter; 2.4698x vs baseline; 1.1096x over previous
import jax
import jax.numpy as jnp
from jax import lax
from jax.experimental import pallas as pl
from jax.experimental.pallas import tpu as pltpu

N_DEV = 4
B = 2
SQ_LOC = 128
D_MODEL = 512
HQ = 16
H_BLK = 4
DH = 64
SKV = 128
WQ_COLS = 256
WO_ROWS = 256


def kernel(x, Wq, K_ext, V_ext, Wo):
    def body(x_ref, wq_ref, k_ref, v_ref, wo_ref, out_ref,
             x_bf, wq_bf, wo_bf, wq_gath, wo_gath, kt_ref, vt_ref,
             send_sems, recv_sems):
        my_pos = lax.axis_index("i")
        left = (my_pos - 1) % N_DEV
        right = (my_pos + 1) % N_DEV
        opp = (my_pos + 2) % N_DEV

        barrier_sem = pltpu.get_barrier_semaphore()
        for nbr in (left, right, opp):
            pl.semaphore_signal(
                barrier_sem, inc=1,
                device_id=(nbr,), device_id_type=pltpu.DeviceIdType.MESH,
            )
        pl.semaphore_wait(barrier_sem, 3)

        wq_bf[...] = wq_ref[...].astype(jnp.bfloat16)
        wo_bf[...] = wo_ref[...].astype(jnp.bfloat16)
        x_bf[...] = x_ref[...].astype(jnp.bfloat16)

        def push(src, gath, target, slot, sem):
            return pltpu.make_async_remote_copy(
                src_ref=src,
                dst_ref=gath.at[slot],
                send_sem=send_sems.at[sem],
                recv_sem=recv_sems.at[sem],
                device_id=(target,),
                device_id_type=pltpu.DeviceIdType.MESH,
            )

        wq_to_r = push(wq_bf, wq_gath, right, 0, 0)
        wq_to_l = push(wq_bf, wq_gath, left, 1, 1)
        wq_to_o = push(wq_bf, wq_gath, opp, 2, 2)
        wo_to_r = push(wo_bf, wo_gath, right, 0, 3)
        wo_to_l = push(wo_bf, wo_gath, left, 1, 4)
        wo_to_o = push(wo_bf, wo_gath, opp, 2, 5)
        for r in (wq_to_r, wo_to_r, wq_to_l, wo_to_l, wq_to_o, wo_to_o):
            r.start()

        for b in range(B):
            for h in range(HQ):
                kt_ref[b, h] = k_ref[b, :, h, :]
                vt_ref[b, h] = v_ref[b, :, h, :]

        rows = lax.broadcasted_iota(jnp.int32, (SQ_LOC, SKV), 0)
        cols = lax.broadcasted_iota(jnp.int32, (SQ_LOC, SKV), 1)
        qb = rows // 64 + 2 * my_pos
        kb = cols // 64
        mask = (qb == kb) | (kb == 0) | ((qb + kb) % 3 == 0)

        def block(origin, wq_c, wo_c):
            partials = []
            for b in range(B):
                qblk = jnp.dot(x_bf[b], wq_c,
                               preferred_element_type=jnp.float32)
                ctxs = []
                for j in range(H_BLK):
                    h = origin * H_BLK + j
                    q = qblk[:, j * DH:(j + 1) * DH]
                    k = kt_ref[b, h]
                    s = lax.dot_general(
                        q, k, (((1,), (1,)), ((), ())),
                        preferred_element_type=jnp.float32,
                    ) * 0.125
                    s = jnp.where(mask, s, -1e9)
                    m = jnp.max(s, axis=1, keepdims=True)
                    w = jnp.exp(s - m)
                    w = w / jnp.sum(w, axis=1, keepdims=True)
                    ctxs.append(jnp.dot(w, vt_ref[b, h],
                                        preferred_element_type=jnp.float32))
                ctx = jnp.concatenate(ctxs, axis=1)
                partials.append(jnp.dot(ctx.astype(jnp.bfloat16), wo_c,
                                        preferred_element_type=jnp.float32))
            return partials

        acc = block(my_pos, wq_bf[...], wo_bf[...])

        wq_to_r.wait_recv()
        wo_to_r.wait_recv()
        p = block(left, wq_gath[0], wo_gath[0])
        acc = [a + q for a, q in zip(acc, p)]

        wq_to_l.wait_recv()
        wo_to_l.wait_recv()
        p = block(right, wq_gath[1], wo_gath[1])
        acc = [a + q for a, q in zip(acc, p)]

        wq_to_o.wait_recv()
        wo_to_o.wait_recv()
        p = block(opp, wq_gath[2], wo_gath[2])
        acc = [a + q for a, q in zip(acc, p)]

        for b in range(B):
            out_ref[b] = acc[b]

        for r in (wq_to_r, wq_to_l, wq_to_o, wo_to_r, wo_to_l, wo_to_o):
            r.wait_send()

    return pl.pallas_call(
        body,
        out_shape=jax.ShapeDtypeStruct((B, SQ_LOC, D_MODEL), jnp.float32),
        in_specs=[pl.BlockSpec(memory_space=pltpu.VMEM)] * 5,
        out_specs=pl.BlockSpec(memory_space=pltpu.VMEM),
        scratch_shapes=[
            pltpu.VMEM((B, SQ_LOC, D_MODEL), jnp.bfloat16),
            pltpu.VMEM((D_MODEL, WQ_COLS), jnp.bfloat16),
            pltpu.VMEM((WO_ROWS, D_MODEL), jnp.bfloat16),
            pltpu.VMEM((3, D_MODEL, WQ_COLS), jnp.bfloat16),
            pltpu.VMEM((3, WO_ROWS, D_MODEL), jnp.bfloat16),
            pltpu.VMEM((B, HQ, SKV, DH), jnp.float32),
            pltpu.VMEM((B, HQ, SKV, DH), jnp.float32),
            pltpu.SemaphoreType.DMA((6,)),
            pltpu.SemaphoreType.DMA((6,)),
        ],
        compiler_params=pltpu.CompilerParams(collective_id=0),
    )(x, Wq, K_ext, V_ext, Wo)


# device time: 20959 ns/iter; 2.8315x vs baseline; 1.1464x over previous
import jax
import jax.numpy as jnp
from jax import lax
from jax.experimental import pallas as pl
from jax.experimental.pallas import tpu as pltpu

N_DEV = 4
B = 2
SQ_LOC = 128
D_MODEL = 512
HQ = 16
H_BLK = 4
DH = 64
SKV = 128
WQ_COLS = 256
WO_ROWS = 256


def kernel(x, Wq, K_ext, V_ext, Wo):
    def body(x_ref, wq_ref, k_ref, v_ref, wo_ref, out_ref,
             x_bf, wq_bf, wo_bf, wq_gath, wo_gath, kt_ref, vt_ref,
             send_sems, recv_sems):
        my_pos = lax.axis_index("i")
        left = (my_pos - 1) % N_DEV
        right = (my_pos + 1) % N_DEV
        opp = (my_pos + 2) % N_DEV

        barrier_sem = pltpu.get_barrier_semaphore()
        for nbr in (opp, left, right):
            pl.semaphore_signal(
                barrier_sem, inc=1,
                device_id=(nbr,), device_id_type=pltpu.DeviceIdType.MESH,
            )

        wq_bf[...] = wq_ref[...].astype(jnp.bfloat16)
        wo_bf[...] = wo_ref[...].astype(jnp.bfloat16)
        x_bf[...] = x_ref[...].astype(jnp.bfloat16)

        pl.semaphore_wait(barrier_sem, 3)

        def push(src, gath, target, slot, sem):
            return pltpu.make_async_remote_copy(
                src_ref=src,
                dst_ref=gath.at[slot],
                send_sem=send_sems.at[sem],
                recv_sem=recv_sems.at[sem],
                device_id=(target,),
                device_id_type=pltpu.DeviceIdType.MESH,
            )

        wq_to_r = push(wq_bf, wq_gath, right, 0, 0)
        wq_to_l = push(wq_bf, wq_gath, left, 1, 1)
        wq_to_o = push(wq_bf, wq_gath, opp, 2, 2)
        wo_to_r = push(wo_bf, wo_gath, right, 0, 3)
        wo_to_l = push(wo_bf, wo_gath, left, 1, 4)
        wo_to_o = push(wo_bf, wo_gath, opp, 2, 5)
        for r in (wq_to_r, wo_to_r, wq_to_l, wo_to_l, wq_to_o, wo_to_o):
            r.start()

        for b in range(B):
            for h in range(HQ):
                kt_ref[b, h] = k_ref[b, :, h, :]
                vt_ref[b, h] = v_ref[b, :, h, :]

        rows = lax.broadcasted_iota(jnp.int32, (SQ_LOC, SKV), 0)
        cols = lax.broadcasted_iota(jnp.int32, (SQ_LOC, SKV), 1)
        qb = rows // 64 + 2 * my_pos
        kb = cols // 64
        mask = (qb == kb) | (kb == 0) | ((qb + kb) % 3 == 0)

        def ctx_stage(origin, wq_c):
            ctxs_b = []
            for b in range(B):
                qblk = jnp.dot(x_bf[b], wq_c,
                               preferred_element_type=jnp.float32)
                parts = []
                for j in range(H_BLK):
                    h = origin * H_BLK + j
                    q = qblk[:, j * DH:(j + 1) * DH]
                    k = kt_ref[b, h]
                    s = lax.dot_general(
                        q, k, (((1,), (1,)), ((), ())),
                        preferred_element_type=jnp.float32,
                    ) * 0.125
                    w = jnp.exp(jnp.where(mask, s, -1e9))
                    d = jnp.sum(w, axis=1, keepdims=True)
                    c = jnp.dot(w, vt_ref[b, h],
                                preferred_element_type=jnp.float32) / d
                    parts.append(c)
                ctxs_b.append(
                    jnp.concatenate(parts, axis=1).astype(jnp.bfloat16))
            return ctxs_b

        def out_stage(ctxs_b, wo_c):
            return [jnp.dot(c, wo_c, preferred_element_type=jnp.float32)
                    for c in ctxs_b]

        acc = out_stage(ctx_stage(my_pos, wq_bf[...]), wo_bf[...])

        wq_to_r.wait_recv()
        c = ctx_stage(left, wq_gath[0])
        wo_to_r.wait_recv()
        acc = [a + q for a, q in zip(acc, out_stage(c, wo_gath[0]))]

        wq_to_l.wait_recv()
        c = ctx_stage(right, wq_gath[1])
        wo_to_l.wait_recv()
        acc = [a + q for a, q in zip(acc, out_stage(c, wo_gath[1]))]

        wq_to_o.wait_recv()
        c = ctx_stage(opp, wq_gath[2])
        wo_to_o.wait_recv()
        acc = [a + q for a, q in zip(acc, out_stage(c, wo_gath[2]))]

        for b in range(B):
            out_ref[b] = acc[b]

        for r in (wq_to_r, wq_to_l, wq_to_o, wo_to_r, wo_to_l, wo_to_o):
            r.wait_send()

    return pl.pallas_call(
        body,
        out_shape=jax.ShapeDtypeStruct((B, SQ_LOC, D_MODEL), jnp.float32),
        in_specs=[pl.BlockSpec(memory_space=pltpu.VMEM)] * 5,
        out_specs=pl.BlockSpec(memory_space=pltpu.VMEM),
        scratch_shapes=[
            pltpu.VMEM((B, SQ_LOC, D_MODEL), jnp.bfloat16),
            pltpu.VMEM((D_MODEL, WQ_COLS), jnp.bfloat16),
            pltpu.VMEM((WO_ROWS, D_MODEL), jnp.bfloat16),
            pltpu.VMEM((3, D_MODEL, WQ_COLS), jnp.bfloat16),
            pltpu.VMEM((3, WO_ROWS, D_MODEL), jnp.bfloat16),
            pltpu.VMEM((B, HQ, SKV, DH), jnp.float32),
            pltpu.VMEM((B, HQ, SKV, DH), jnp.float32),
            pltpu.SemaphoreType.DMA((6,)),
            pltpu.SemaphoreType.DMA((6,)),
        ],
        compiler_params=pltpu.CompilerParams(collective_id=0),
    )(x, Wq, K_ext, V_ext, Wo)


# device time: 17267 ns/iter; 3.4369x vs baseline; 1.2138x over previous
import jax
import jax.numpy as jnp
from jax import lax
from jax.experimental import pallas as pl
from jax.experimental.pallas import tpu as pltpu

N_DEV = 4
B = 2
SQ_LOC = 128
D_MODEL = 512
HQ = 16
H_BLK = 4
DH = 64
SKV = 128
WQ_COLS = 256
WO_ROWS = 256


def kernel(x, Wq, K_ext, V_ext, Wo):
    def body(x_ref, wq_ref, k_ref, v_ref, wo_ref, out_ref,
             x_bf, wq_i8, wo_i8, sc_snd, wq_gath, wo_gath, sc_gath,
             kt_ref, vt_ref, send_sems, recv_sems):
        my_pos = lax.axis_index("i")
        left = (my_pos - 1) % N_DEV
        right = (my_pos + 1) % N_DEV
        opp = (my_pos + 2) % N_DEV

        barrier_sem = pltpu.get_barrier_semaphore()
        for nbr in (opp, left, right):
            pl.semaphore_signal(
                barrier_sem, inc=1,
                device_id=(nbr,), device_id_type=pltpu.DeviceIdType.MESH,
            )

        wq_f = wq_ref[...]
        wo_f = wo_ref[...]
        wq_amax = jnp.maximum(jnp.max(jnp.abs(wq_f)), 1e-20)
        wo_amax = jnp.maximum(jnp.max(jnp.abs(wo_f)), 1e-20)
        wq_i8[...] = jnp.round(wq_f * (127.0 / wq_amax)).astype(jnp.int8)
        wo_i8[...] = jnp.round(wo_f * (127.0 / wo_amax)).astype(jnp.int8)
        sc_snd[0:1, :] = jnp.full((1, 128), wq_amax / 127.0, jnp.float32)
        sc_snd[1:2, :] = jnp.full((1, 128), wo_amax / 127.0, jnp.float32)
        x_bf[...] = x_ref[...].astype(jnp.bfloat16)

        pl.semaphore_wait(barrier_sem, 3)

        def push(src, gath, target, slot, sem):
            return pltpu.make_async_remote_copy(
                src_ref=src,
                dst_ref=gath.at[slot],
                send_sem=send_sems.at[sem],
                recv_sem=recv_sems.at[sem],
                device_id=(target,),
                device_id_type=pltpu.DeviceIdType.MESH,
            )

        sc_to_r = push(sc_snd, sc_gath, right, 0, 0)
        sc_to_l = push(sc_snd, sc_gath, left, 1, 1)
        sc_to_o = push(sc_snd, sc_gath, opp, 2, 2)
        wq_to_r = push(wq_i8, wq_gath, right, 0, 3)
        wq_to_l = push(wq_i8, wq_gath, left, 1, 4)
        wq_to_o = push(wq_i8, wq_gath, opp, 2, 5)
        wo_to_r = push(wo_i8, wo_gath, right, 0, 6)
        wo_to_l = push(wo_i8, wo_gath, left, 1, 7)
        wo_to_o = push(wo_i8, wo_gath, opp, 2, 8)
        for r in (sc_to_r, sc_to_l, sc_to_o,
                  wq_to_r, wo_to_r, wq_to_l, wo_to_l, wq_to_o, wo_to_o):
            r.start()

        for b in range(B):
            for h in range(HQ):
                kt_ref[b, h] = k_ref[b, :, h, :]
                vt_ref[b, h] = v_ref[b, :, h, :]

        rows = lax.broadcasted_iota(jnp.int32, (SQ_LOC, SKV), 0)
        cols = lax.broadcasted_iota(jnp.int32, (SQ_LOC, SKV), 1)
        qb = rows // 64 + 2 * my_pos
        kb = cols // 64
        mask = (qb == kb) | (kb == 0) | ((qb + kb) % 3 == 0)

        def ctx_stage(origin, wq_c, score_scale):
            ctxs_b = []
            for b in range(B):
                qblk = jnp.dot(x_bf[b], wq_c,
                               preferred_element_type=jnp.float32)
                parts = []
                for j in range(H_BLK):
                    h = origin * H_BLK + j
                    q = qblk[:, j * DH:(j + 1) * DH]
                    k = kt_ref[b, h]
                    s = lax.dot_general(
                        q, k, (((1,), (1,)), ((), ())),
                        preferred_element_type=jnp.float32,
                    ) * score_scale
                    w = jnp.exp(jnp.where(mask, s, -1e9))
                    d = jnp.sum(w, axis=1, keepdims=True)
                    c = jnp.dot(w, vt_ref[b, h],
                                preferred_element_type=jnp.float32) / d
                    parts.append(c)
                ctxs_b.append(
                    jnp.concatenate(parts, axis=1).astype(jnp.bfloat16))
            return ctxs_b

        def out_stage(ctxs_b, wo_c, out_scale=None):
            ps = [jnp.dot(c, wo_c, preferred_element_type=jnp.float32)
                  for c in ctxs_b]
            if out_scale is not None:
                ps = [p * out_scale for p in ps]
            return ps

        acc = out_stage(ctx_stage(my_pos, wq_ref[...].astype(jnp.bfloat16),
                                  0.125),
                        wo_ref[...].astype(jnp.bfloat16))

        def peer_block(origin, slot, sc_rdma, wq_rdma, wo_rdma):
            sc_rdma.wait_recv()
            wq_rdma.wait_recv()
            s_wq = sc_gath[slot, 0, 0]
            s_wo = sc_gath[slot, 1, 0]
            c = ctx_stage(origin, wq_gath[slot].astype(jnp.bfloat16),
                          0.125 * s_wq)
            wo_rdma.wait_recv()
            return out_stage(c, wo_gath[slot].astype(jnp.bfloat16), s_wo)

        p = peer_block(left, 0, sc_to_r, wq_to_r, wo_to_r)
        acc = [a + q for a, q in zip(acc, p)]
        p = peer_block(right, 1, sc_to_l, wq_to_l, wo_to_l)
        acc = [a + q for a, q in zip(acc, p)]
        p = peer_block(opp, 2, sc_to_o, wq_to_o, wo_to_o)
        acc = [a + q for a, q in zip(acc, p)]

        for b in range(B):
            out_ref[b] = acc[b]

        for r in (sc_to_r, sc_to_l, sc_to_o, wq_to_r, wq_to_l, wq_to_o,
                  wo_to_r, wo_to_l, wo_to_o):
            r.wait_send()

    return pl.pallas_call(
        body,
        out_shape=jax.ShapeDtypeStruct((B, SQ_LOC, D_MODEL), jnp.float32),
        in_specs=[pl.BlockSpec(memory_space=pltpu.VMEM)] * 5,
        out_specs=pl.BlockSpec(memory_space=pltpu.VMEM),
        scratch_shapes=[
            pltpu.VMEM((B, SQ_LOC, D_MODEL), jnp.bfloat16),
            pltpu.VMEM((D_MODEL, WQ_COLS), jnp.int8),
            pltpu.VMEM((WO_ROWS, D_MODEL), jnp.int8),
            pltpu.VMEM((2, 128), jnp.float32),
            pltpu.VMEM((3, D_MODEL, WQ_COLS), jnp.int8),
            pltpu.VMEM((3, WO_ROWS, D_MODEL), jnp.int8),
            pltpu.VMEM((3, 2, 128), jnp.float32),
            pltpu.VMEM((B, HQ, SKV, DH), jnp.float32),
            pltpu.VMEM((B, HQ, SKV, DH), jnp.float32),
            pltpu.SemaphoreType.DMA((9,)),
            pltpu.SemaphoreType.DMA((9,)),
        ],
        compiler_params=pltpu.CompilerParams(collective_id=0),
    )(x, Wq, K_ext, V_ext, Wo)
